# baseline (speedup 1.0000x reference)
import functools

import jax
import jax.numpy as jnp
from jax import lax
from jax.experimental import pallas as pl
from jax.experimental.pallas import tpu as pltpu
from jax.experimental.pallas import tpu_sc as plsc

_B, _S, _D = 4, 2048, 1024
_NW = 32
_ROWS_PER_W = _S // _NW
_CH = 32
_CHW = _CH * _D
_LANES = 16
_INNER = 64


def _tc_add_body(x_ref, pos_ref, out_ref):
    out_ref[...] = x_ref[...] + pos_ref[...][None, :, :]


def _tc_add(x, pos_table):
    B, S, D = x.shape
    S_BLK = 512
    return pl.pallas_call(
        _tc_add_body,
        grid=(S // S_BLK,),
        in_specs=[
            pl.BlockSpec((B, S_BLK, D), lambda i: (0, i, 0)),
            pl.BlockSpec((S_BLK, D), lambda i: (i, 0)),
        ],
        out_specs=pl.BlockSpec((B, S_BLK, D), lambda i: (0, i, 0)),
        out_shape=jax.ShapeDtypeStruct((B, S, D), x.dtype),
    )(x, pos_table)


def _sc_body(x_hbm, pos_hbm, out_hbm, posbuf, xbuf):
    wid = lax.axis_index("s") * 2 + lax.axis_index("c")
    for c in range(_ROWS_PER_W // _CH):
        seq0 = wid * _ROWS_PER_W + c * _CH
        pltpu.sync_copy(pos_hbm.at[pl.ds(seq0 * _D, _CHW)], posbuf)
        for b in range(_B):
            off = (b * _S) * _D + seq0 * _D
            pltpu.sync_copy(x_hbm.at[pl.ds(off, _CHW)], xbuf)

            def body(i, _):
                base = i * (_INNER * _LANES)
                for j in range(_INNER):
                    sl = pl.ds(base + j * _LANES, _LANES)
                    xbuf[sl] = xbuf[sl] + posbuf[sl]
                return 0

            lax.fori_loop(0, _CHW // (_INNER * _LANES), body, 0)
            pltpu.sync_copy(xbuf, out_hbm.at[pl.ds(off, _CHW)])


def _sc_add(x, pos_table):
    mesh = plsc.VectorSubcoreMesh(core_axis_name="c", subcore_axis_name="s")
    call = functools.partial(
        pl.kernel,
        mesh=mesh,
        out_type=jax.ShapeDtypeStruct((_B * _S * _D,), jnp.float32),
        scratch_types=[
            pltpu.VMEM((_CHW,), jnp.float32),
            pltpu.VMEM((_CHW,), jnp.float32),
        ],
    )(_sc_body)
    out = call(x.reshape(-1), pos_table.reshape(-1))
    return out.reshape(_B, _S, _D)


def kernel(x, pos_table):
    return _sc_add(x, pos_table)


# speedup vs baseline: 1.1964x; 1.1964x over previous
import functools

import jax
import jax.numpy as jnp
from jax import lax
from jax.experimental import pallas as pl
from jax.experimental.pallas import tpu as pltpu
from jax.experimental.pallas import tpu_sc as plsc

_B, _S, _D = 4, 2048, 1024
_NW = 32
_ROWS_PER_W = _S // _NW
_CH = 16
_CHW = _CH * _D
_LANES = 16
_INNER = 64


def _tc_add_body(x_ref, pos_ref, out_ref):
    out_ref[...] = x_ref[...] + pos_ref[...][None, :, :]


def _tc_add(x, pos_table):
    B, S, D = x.shape
    S_BLK = 512
    return pl.pallas_call(
        _tc_add_body,
        grid=(S // S_BLK,),
        in_specs=[
            pl.BlockSpec((B, S_BLK, D), lambda i: (0, i, 0)),
            pl.BlockSpec((S_BLK, D), lambda i: (i, 0)),
        ],
        out_specs=pl.BlockSpec((B, S_BLK, D), lambda i: (0, i, 0)),
        out_shape=jax.ShapeDtypeStruct((B, S, D), x.dtype),
    )(x, pos_table)


def _sc_body(x_hbm, pos_hbm, out_hbm,
             pbuf0, pbuf1, xbuf0, xbuf1, xbuf2,
             psem0, psem1, lsem0, lsem1, lsem2, ssem0, ssem1, ssem2):
    wid = lax.axis_index("s") * 2 + lax.axis_index("c")
    pbufs, psems = [pbuf0, pbuf1], [psem0, psem1]
    xbufs = [xbuf0, xbuf1, xbuf2]
    lsems = [lsem0, lsem1, lsem2]
    ssems = [ssem0, ssem1, ssem2]
    n_ch = _ROWS_PER_W // _CH

    def x_off(c, b):
        return (b * _S) * _D + (wid * _ROWS_PER_W + c * _CH) * _D

    jobs = [(c, b) for c in range(n_ch) for b in range(_B)]
    n = len(jobs)

    pos_h = [None] * n_ch
    pos_h[0] = pltpu.async_copy(
        pos_hbm.at[pl.ds(wid * _ROWS_PER_W * _D, _CHW)], pbufs[0], psems[0])
    load_h = [None] * n
    store_h = [None] * n
    for i in range(min(2, n)):
        c, b = jobs[i]
        load_h[i] = pltpu.async_copy(
            x_hbm.at[pl.ds(x_off(c, b), _CHW)], xbufs[i % 3], lsems[i % 3])

    for i in range(n):
        c, b = jobs[i]
        if b == 0:
            if c + 1 < n_ch:
                nc = c + 1
                pos_h[nc] = pltpu.async_copy(
                    pos_hbm.at[pl.ds((wid * _ROWS_PER_W + nc * _CH) * _D, _CHW)],
                    pbufs[nc % 2], psems[nc % 2])
            pos_h[c].wait()
        xb = xbufs[i % 3]
        pb = pbufs[c % 2]
        load_h[i].wait()

        @plsc.parallel_loop(0, _CHW, step=_LANES, unroll=8)
        def _(k):
            sl = pl.ds(k, _LANES)
            xb[sl] = xb[sl] + pb[sl]

        store_h[i] = pltpu.async_copy(
            xb, out_hbm.at[pl.ds(x_off(c, b), _CHW)], ssems[i % 3])
        if i + 2 < n:
            j = i + 2
            if j >= 3:
                store_h[j - 3].wait()
            cj, bj = jobs[j]
            load_h[j] = pltpu.async_copy(
                x_hbm.at[pl.ds(x_off(cj, bj), _CHW)], xbufs[j % 3], lsems[j % 3])
    for i in range(max(0, n - 3), n):
        store_h[i].wait()


def _sc_add(x, pos_table):
    mesh = plsc.VectorSubcoreMesh(core_axis_name="c", subcore_axis_name="s")
    call = functools.partial(
        pl.kernel,
        mesh=mesh,
        out_type=jax.ShapeDtypeStruct((_B * _S * _D,), jnp.float32),
        scratch_types=[
            pltpu.VMEM((_CHW,), jnp.float32),
            pltpu.VMEM((_CHW,), jnp.float32),
            pltpu.VMEM((_CHW,), jnp.float32),
            pltpu.VMEM((_CHW,), jnp.float32),
            pltpu.VMEM((_CHW,), jnp.float32),
            pltpu.SemaphoreType.DMA,
            pltpu.SemaphoreType.DMA,
            pltpu.SemaphoreType.DMA,
            pltpu.SemaphoreType.DMA,
            pltpu.SemaphoreType.DMA,
            pltpu.SemaphoreType.DMA,
            pltpu.SemaphoreType.DMA,
            pltpu.SemaphoreType.DMA,
        ],
    )(_sc_body)
    out = call(x.reshape(-1), pos_table.reshape(-1))
    return out.reshape(_B, _S, _D)


def kernel(x, pos_table):
    return _sc_add(x, pos_table)


# speedup vs baseline: 2.9317x; 2.4505x over previous
import functools

import jax
import jax.numpy as jnp
from jax import lax
from jax.experimental import pallas as pl
from jax.experimental.pallas import tpu as pltpu
from jax.experimental.pallas import tpu_sc as plsc

_B, _S, _D = 4, 2048, 1024
_NW = 32
_ROWS_PER_W = _S // _NW
_CH = 16
_CHW = _CH * _D
_LANES = 16
_GROUPS = _D // _LANES


def _tc_add_body(x_ref, pos_ref, out_ref):
    out_ref[...] = x_ref[...] + pos_ref[...][None, :, :]


def _tc_add(x, pos_table):
    B, S, D = x.shape
    S_BLK = 512
    return pl.pallas_call(
        _tc_add_body,
        grid=(S // S_BLK,),
        in_specs=[
            pl.BlockSpec((B, S_BLK, D), lambda i: (0, i, 0)),
            pl.BlockSpec((S_BLK, D), lambda i: (i, 0)),
        ],
        out_specs=pl.BlockSpec((B, S_BLK, D), lambda i: (0, i, 0)),
        out_shape=jax.ShapeDtypeStruct((B, S, D), x.dtype),
    )(x, pos_table)


def _sc_body(x_hbm, pos_hbm, out_hbm,
             pbuf0, pbuf1, xbuf0, xbuf1, xbuf2,
             psem0, psem1, lsem0, lsem1, lsem2, ssem0, ssem1, ssem2):
    wid = lax.axis_index("s") * 2 + lax.axis_index("c")
    pbufs, psems = [pbuf0, pbuf1], [psem0, psem1]
    xbufs = [xbuf0, xbuf1, xbuf2]
    lsems = [lsem0, lsem1, lsem2]
    ssems = [ssem0, ssem1, ssem2]
    n_ch = _ROWS_PER_W // _CH

    def seq0(c):
        return wid * _ROWS_PER_W + c * _CH

    jobs = [(c, b) for c in range(n_ch) for b in range(_B)]
    n = len(jobs)

    pos_h = [None] * n_ch
    pos_h[0] = pltpu.async_copy(
        pos_hbm.at[pl.ds(seq0(0), _CH), :], pbufs[0], psems[0])
    load_h = [None] * n
    store_h = [None] * n
    for i in range(min(2, n)):
        c, b = jobs[i]
        load_h[i] = pltpu.async_copy(
            x_hbm.at[b, pl.ds(seq0(c), _CH), :], xbufs[i % 3], lsems[i % 3])

    for i in range(n):
        c, b = jobs[i]
        if b == 0:
            if c + 1 < n_ch:
                nc = c + 1
                pos_h[nc] = pltpu.async_copy(
                    pos_hbm.at[pl.ds(seq0(nc), _CH), :],
                    pbufs[nc % 2], psems[nc % 2])
            pos_h[c].wait()
        xb = xbufs[i % 3]
        pb = pbufs[c % 2]
        load_h[i].wait()

        @plsc.parallel_loop(0, _CHW, step=_LANES, unroll=8)
        def _(k):
            r = k // _D
            sl = pl.ds(k % _D, _LANES)
            xb[r, sl] = xb[r, sl] + pb[r, sl]

        store_h[i] = pltpu.async_copy(
            xb, out_hbm.at[b, pl.ds(seq0(c), _CH), :], ssems[i % 3])
        if i + 2 < n:
            j = i + 2
            if j >= 3:
                store_h[j - 3].wait()
            cj, bj = jobs[j]
            load_h[j] = pltpu.async_copy(
                x_hbm.at[bj, pl.ds(seq0(cj), _CH), :], xbufs[j % 3], lsems[j % 3])
    for i in range(max(0, n - 3), n):
        store_h[i].wait()


def _sc_add(x, pos_table):
    mesh = plsc.VectorSubcoreMesh(core_axis_name="c", subcore_axis_name="s")
    call = functools.partial(
        pl.kernel,
        mesh=mesh,
        out_type=jax.ShapeDtypeStruct((_B, _S, _D), jnp.float32),
        scratch_types=[
            pltpu.VMEM((_CH, _D), jnp.float32),
            pltpu.VMEM((_CH, _D), jnp.float32),
            pltpu.VMEM((_CH, _D), jnp.float32),
            pltpu.VMEM((_CH, _D), jnp.float32),
            pltpu.VMEM((_CH, _D), jnp.float32),
            pltpu.SemaphoreType.DMA,
            pltpu.SemaphoreType.DMA,
            pltpu.SemaphoreType.DMA,
            pltpu.SemaphoreType.DMA,
            pltpu.SemaphoreType.DMA,
            pltpu.SemaphoreType.DMA,
            pltpu.SemaphoreType.DMA,
            pltpu.SemaphoreType.DMA,
        ],
    )(_sc_body)
    return call(x, pos_table)


def kernel(x, pos_table):
    return _sc_add(x, pos_table)


# speedup vs baseline: 5.3397x; 1.8214x over previous
import functools

import jax
import jax.numpy as jnp
from jax import lax
from jax.experimental import pallas as pl
from jax.experimental.pallas import tpu as pltpu
from jax.experimental.pallas import tpu_sc as plsc

_B, _S, _D = 4, 2048, 1024
_NW = 32
_ROWS_PER_W = _S // _NW
_CH = 16
_CHW = _CH * _D
_LANES = 16
_GROUPS = _D // _LANES


def _tc_add_body(x_ref, pos_ref, out_ref):
    out_ref[...] = x_ref[...] + pos_ref[...][None, :, :]


def _tc_add(x, pos_table):
    B, S, D = x.shape
    S_BLK = 512
    return pl.pallas_call(
        _tc_add_body,
        grid=(S // S_BLK, B),
        in_specs=[
            pl.BlockSpec((1, S_BLK, D), lambda i, b: (b, i, 0)),
            pl.BlockSpec((S_BLK, D), lambda i, b: (i, 0)),
        ],
        out_specs=pl.BlockSpec((1, S_BLK, D), lambda i, b: (b, i, 0)),
        out_shape=jax.ShapeDtypeStruct((B, S, D), x.dtype),
    )(x, pos_table)


def _sc_body(x_hbm, pos_hbm, out_hbm,
             pbuf0, pbuf1, xbuf0, xbuf1, xbuf2,
             psem0, psem1, lsem0, lsem1, lsem2, ssem0, ssem1, ssem2):
    wid = lax.axis_index("s") * 2 + lax.axis_index("c")
    pbufs, psems = [pbuf0, pbuf1], [psem0, psem1]
    xbufs = [xbuf0, xbuf1, xbuf2]
    lsems = [lsem0, lsem1, lsem2]
    ssems = [ssem0, ssem1, ssem2]
    n_ch = _ROWS_PER_W // _CH

    def seq0(c):
        return wid * _ROWS_PER_W + c * _CH

    jobs = [(c, b) for c in range(n_ch) for b in range(_B)]
    n = len(jobs)

    pos_h = [None] * n_ch
    pos_h[0] = pltpu.async_copy(
        pos_hbm.at[pl.ds(seq0(0), _CH), :], pbufs[0], psems[0])
    load_h = [None] * n
    store_h = [None] * n
    for i in range(min(2, n)):
        c, b = jobs[i]
        load_h[i] = pltpu.async_copy(
            x_hbm.at[b, pl.ds(seq0(c), _CH), :], xbufs[i % 3], lsems[i % 3])

    for i in range(n):
        c, b = jobs[i]
        if b == 0:
            if c + 1 < n_ch:
                nc = c + 1
                pos_h[nc] = pltpu.async_copy(
                    pos_hbm.at[pl.ds(seq0(nc), _CH), :],
                    pbufs[nc % 2], psems[nc % 2])
            pos_h[c].wait()
        xb = xbufs[i % 3]
        pb = pbufs[c % 2]
        load_h[i].wait()

        @plsc.parallel_loop(0, _CHW, step=_LANES, unroll=8)
        def _(k):
            r = k // _D
            sl = pl.ds(k % _D, _LANES)
            xb[r, sl] = xb[r, sl] + pb[r, sl]

        store_h[i] = pltpu.async_copy(
            xb, out_hbm.at[b, pl.ds(seq0(c), _CH), :], ssems[i % 3])
        if i + 2 < n:
            j = i + 2
            if j >= 3:
                store_h[j - 3].wait()
            cj, bj = jobs[j]
            load_h[j] = pltpu.async_copy(
                x_hbm.at[bj, pl.ds(seq0(cj), _CH), :], xbufs[j % 3], lsems[j % 3])
    for i in range(max(0, n - 3), n):
        store_h[i].wait()


def _sc_add(x, pos_table):
    mesh = plsc.VectorSubcoreMesh(core_axis_name="c", subcore_axis_name="s")
    call = functools.partial(
        pl.kernel,
        mesh=mesh,
        out_type=jax.ShapeDtypeStruct((_B, _S, _D), jnp.float32),
        scratch_types=[
            pltpu.VMEM((_CH, _D), jnp.float32),
            pltpu.VMEM((_CH, _D), jnp.float32),
            pltpu.VMEM((_CH, _D), jnp.float32),
            pltpu.VMEM((_CH, _D), jnp.float32),
            pltpu.VMEM((_CH, _D), jnp.float32),
            pltpu.SemaphoreType.DMA,
            pltpu.SemaphoreType.DMA,
            pltpu.SemaphoreType.DMA,
            pltpu.SemaphoreType.DMA,
            pltpu.SemaphoreType.DMA,
            pltpu.SemaphoreType.DMA,
            pltpu.SemaphoreType.DMA,
            pltpu.SemaphoreType.DMA,
        ],
    )(_sc_body)
    return call(x, pos_table)


def kernel(x, pos_table):
    return _tc_add(x, pos_table)
